# baseline (speedup 1.0000x reference)
import functools

import jax
import jax.numpy as jnp
from jax.experimental import pallas as pl
from jax.experimental.pallas import tpu as pltpu

_ALPHA = 0.2
_PREC = jax.lax.Precision.DEFAULT


def _main_kernel(adj_ref, x_hbm, wbm_hbm, z_ref, q_ref,
                 xv, wbmv, sup_ref, sem, *, nfeat, nclust):
    @pl.when(pl.program_id(0) == 0)
    def _():
        cx = pltpu.make_async_copy(x_hbm, xv, sem)
        cx.start()
        cx.wait()
        cw = pltpu.make_async_copy(wbm_hbm, wbmv, sem)
        cw.start()
        cw.wait()
        sup = jnp.dot(xv[...], wbmv[:nfeat, :],
                      preferred_element_type=jnp.float32,
                      precision=_PREC)
        sup_ref[...] = sup.astype(jnp.bfloat16)

    z = jnp.dot(adj_ref[...].astype(jnp.bfloat16), sup_ref[...],
                preferred_element_type=jnp.float32,
                precision=_PREC) + wbmv[nfeat:nfeat + 1, :]
    z_ref[...] = z
    mu = wbmv[nfeat + 1:nfeat + 1 + nclust, :]
    zsq = jnp.sum(z * z, axis=1, keepdims=True)
    musq = jnp.sum(mu * mu, axis=1)
    cross = jax.lax.dot_general(
        z, mu, dimension_numbers=(((1,), (1,)), ((), ())),
        preferred_element_type=jnp.float32, precision=_PREC)
    d2 = zsq + musq[None, :] - 2.0 * cross
    q = 1.0 / (1.0 + d2 / _ALPHA + 1e-8)
    q = q ** (_ALPHA + 1.0)
    q_ref[...] = q / jnp.sum(q, axis=1, keepdims=True)


def kernel(x, adj, W, b, mu):
    n, nfeat = x.shape
    nhid = W.shape[1]
    nclust = mu.shape[0]

    wbm = jnp.concatenate([W, b.reshape(1, nhid), mu], axis=0)

    bm = 400
    z, q = pl.pallas_call(
        functools.partial(_main_kernel, nfeat=nfeat, nclust=nclust),
        grid=(n // bm,),
        in_specs=[
            pl.BlockSpec((bm, n), lambda i: (i, 0)),
            pl.BlockSpec(memory_space=pl.ANY),
            pl.BlockSpec(memory_space=pl.ANY),
        ],
        out_specs=[
            pl.BlockSpec((bm, nhid), lambda i: (i, 0)),
            pl.BlockSpec((bm, nclust), lambda i: (i, 0)),
        ],
        out_shape=[
            jax.ShapeDtypeStruct((n, nhid), jnp.float32),
            jax.ShapeDtypeStruct((n, nclust), jnp.float32),
        ],
        scratch_shapes=[
            pltpu.VMEM((n, nfeat), jnp.float32),
            pltpu.VMEM((nfeat + 1 + nclust, nhid), jnp.float32),
            pltpu.VMEM((n, nhid), jnp.bfloat16),
            pltpu.SemaphoreType.DMA,
        ],
        compiler_params=pltpu.CompilerParams(
            dimension_semantics=("arbitrary",)),
    )(adj, x, wbm)
    return z, q


# speedup vs baseline: 1.0345x; 1.0345x over previous
import functools

import jax
import jax.numpy as jnp
from jax.experimental import pallas as pl
from jax.experimental.pallas import tpu as pltpu

_ALPHA = 0.2
_PREC = jax.lax.Precision.DEFAULT


def _soft_assign(z, mu):
    zsq = jnp.sum(z * z, axis=1, keepdims=True)
    musq = jnp.sum(mu * mu, axis=1)
    cross = jax.lax.dot_general(
        z, mu, dimension_numbers=(((1,), (1,)), ((), ())),
        preferred_element_type=jnp.float32, precision=_PREC)
    d2 = zsq + musq[None, :] - 2.0 * cross
    q = 1.0 / (1.0 + d2 / _ALPHA + 1e-8)
    q = q ** (_ALPHA + 1.0)
    return q / jnp.sum(q, axis=1, keepdims=True)


def _main_kernel(adj_hbm, x_ref, wbm_ref, z_ref, q_ref,
                 bufs, sup_ref, sems, *, bm, nblk, nfeat, nclust):
    def adj_copy(blk, slot):
        return pltpu.make_async_copy(
            adj_hbm.at[pl.ds(blk * bm, bm), :], bufs.at[slot], sems.at[slot])

    adj_copy(0, 0).start()
    adj_copy(1, 1).start()

    sup = jnp.dot(x_ref[...], wbm_ref[:nfeat, :],
                  preferred_element_type=jnp.float32, precision=_PREC)
    sup_ref[...] = sup.astype(jnp.bfloat16)

    b = wbm_ref[nfeat:nfeat + 1, :]
    mu = wbm_ref[nfeat + 1:nfeat + 1 + nclust, :]

    def body(i, carry):
        slot = jax.lax.rem(i, 2)
        adj_copy(i, slot).wait()
        z = jnp.dot(bufs[slot].astype(jnp.bfloat16), sup_ref[...],
                    preferred_element_type=jnp.float32,
                    precision=_PREC) + b

        @pl.when(i + 2 < nblk)
        def _():
            adj_copy(i + 2, slot).start()

        z_ref[pl.ds(i * bm, bm), :] = z
        q_ref[pl.ds(i * bm, bm), :] = _soft_assign(z, mu)
        return carry

    jax.lax.fori_loop(0, nblk, body, 0)


def kernel(x, adj, W, b, mu):
    n, nfeat = x.shape
    nhid = W.shape[1]
    nclust = mu.shape[0]

    wbm = jnp.concatenate([W, b.reshape(1, nhid), mu], axis=0)

    bm = 400
    nblk = n // bm
    z, q = pl.pallas_call(
        functools.partial(_main_kernel, bm=bm, nblk=nblk, nfeat=nfeat,
                          nclust=nclust),
        grid=(1,),
        in_specs=[
            pl.BlockSpec(memory_space=pl.ANY),
            pl.BlockSpec((n, nfeat), lambda i: (0, 0)),
            pl.BlockSpec((nfeat + 1 + nclust, nhid), lambda i: (0, 0)),
        ],
        out_specs=[
            pl.BlockSpec((n, nhid), lambda i: (0, 0)),
            pl.BlockSpec((n, nclust), lambda i: (0, 0)),
        ],
        out_shape=[
            jax.ShapeDtypeStruct((n, nhid), jnp.float32),
            jax.ShapeDtypeStruct((n, nclust), jnp.float32),
        ],
        scratch_shapes=[
            pltpu.VMEM((2, bm, n), jnp.float32),
            pltpu.VMEM((n, nhid), jnp.bfloat16),
            pltpu.SemaphoreType.DMA((2,)),
        ],
        compiler_params=pltpu.CompilerParams(
            dimension_semantics=("arbitrary",)),
    )(adj, x, wbm)
    return z, q


# speedup vs baseline: 1.0521x; 1.0170x over previous
import jax
import jax.numpy as jnp
from jax.experimental import pallas as pl
from jax.experimental.pallas import tpu as pltpu

_ALPHA = 0.2
_PREC = jax.lax.Precision.DEFAULT


def _main_kernel(adj_ref, x_ref, w_ref, b_ref, mu_ref, z_ref, q_ref,
                 sup_ref):
    @pl.when(pl.program_id(0) == 0)
    def _():
        sup = jnp.dot(x_ref[...], w_ref[...],
                      preferred_element_type=jnp.float32,
                      precision=_PREC)
        sup_ref[...] = sup.astype(jnp.bfloat16)

    adj_bf = adj_ref[...].astype(jnp.bfloat16)
    z = jnp.dot(adj_bf, sup_ref[...],
                preferred_element_type=jnp.float32,
                precision=_PREC) + b_ref[...]
    z_ref[...] = z
    mu = mu_ref[...]
    zsq = jnp.sum(z * z, axis=1, keepdims=True)
    musq = jnp.sum(mu * mu, axis=1)
    cross = jax.lax.dot_general(
        z, mu, dimension_numbers=(((1,), (1,)), ((), ())),
        preferred_element_type=jnp.float32, precision=_PREC)
    d2 = zsq + musq[None, :] - 2.0 * cross
    q = 1.0 / (1.0 + d2 / _ALPHA + 1e-8)
    q = q ** (_ALPHA + 1.0)
    q_ref[...] = q / jnp.sum(q, axis=1, keepdims=True)


def kernel(x, adj, W, b, mu):
    n, nfeat = x.shape
    nhid = W.shape[1]
    nclust = mu.shape[0]

    bm = 400
    z, q = pl.pallas_call(
        _main_kernel,
        grid=(n // bm,),
        in_specs=[
            pl.BlockSpec((bm, n), lambda i: (i, 0)),
            pl.BlockSpec((n, nfeat), lambda i: (0, 0),
                         pipeline_mode=pl.Buffered(buffer_count=1)),
            pl.BlockSpec((nfeat, nhid), lambda i: (0, 0)),
            pl.BlockSpec((1, nhid), lambda i: (0, 0)),
            pl.BlockSpec((nclust, nhid), lambda i: (0, 0)),
        ],
        out_specs=[
            pl.BlockSpec((bm, nhid), lambda i: (i, 0)),
            pl.BlockSpec((bm, nclust), lambda i: (i, 0)),
        ],
        out_shape=[
            jax.ShapeDtypeStruct((n, nhid), jnp.float32),
            jax.ShapeDtypeStruct((n, nclust), jnp.float32),
        ],
        scratch_shapes=[pltpu.VMEM((n, nhid), jnp.bfloat16)],
        compiler_params=pltpu.CompilerParams(
            dimension_semantics=("arbitrary",)),
    )(adj, x, W, b.reshape(1, nhid), mu)
    return z, q
